# baseline (speedup 1.0000x reference)
import functools

import jax
import jax.numpy as jnp
from jax import lax
from jax.experimental import pallas as pl
from jax.experimental.pallas import tpu as pltpu
from jax.experimental.pallas import tpu_sc as plsc

B, S = 4, 8192
V, D = 256, 1024
N = B * S

NC, NS = 2, 16
NW = NC * NS
ROWS_PER_W = N // NW
C = 32
NCHUNK = ROWS_PER_W // C

_mesh = plsc.VectorSubcoreMesh(core_axis_name="c", subcore_axis_name="s")


@functools.partial(
    pl.kernel,
    mesh=_mesh,
    out_type=jax.ShapeDtypeStruct((N, D), jnp.float32),
    scratch_types=[
        pltpu.VMEM((NCHUNK, C), jnp.int32),
        pltpu.VMEM((C, D), jnp.float32),
        pltpu.VMEM((C, D), jnp.float32),
        pltpu.SemaphoreType.DMA,
        pltpu.SemaphoreType.DMA,
    ],
)
def _sc_gather(idx_hbm, table_hbm, out_hbm, idx_v, rows0, rows1,
               sem0, sem1):
    wid = lax.axis_index("s") * NC + lax.axis_index("c")
    pltpu.sync_copy(idx_hbm.at[wid], idx_v)

    base = wid * ROWS_PER_W
    bufs = (rows0, rows1)
    sems = (sem0, sem1)

    pltpu.async_copy(table_hbm.at[idx_v.at[0]], rows0, sem0)

    def outer(i2, carry):
        c0 = i2 * 2
        for b in range(2):
            c = c0 + b
            nxt = bufs[1 - b]
            nxt_sem = sems[1 - b]

            @pl.when(c + 1 < NCHUNK)
            def _():
                pltpu.async_copy(table_hbm.at[idx_v.at[c + 1]], nxt, nxt_sem)

            pltpu.make_async_copy(table_hbm.at[idx_v.at[c]], bufs[b], sems[b]).wait()
            pltpu.sync_copy(bufs[b], out_hbm.at[pl.ds(base + c * C, C)])
        return carry

    lax.fori_loop(0, NCHUNK // 2, outer, 0)


def kernel(input_ids, attention_mask, embed):
    idx = input_ids.reshape(NW, NCHUNK, C).astype(jnp.int32)
    out = _sc_gather(idx, embed)
    return out.reshape(B, S, D)


# speedup vs baseline: 1.4341x; 1.4341x over previous
import functools

import jax
import jax.numpy as jnp
from jax import lax
from jax.experimental import pallas as pl
from jax.experimental.pallas import tpu as pltpu
from jax.experimental.pallas import tpu_sc as plsc

B, S = 4, 8192
V, D = 256, 1024
N = B * S

NC, NS = 2, 16
NW = NC * NS
ROWS_PER_W = N // NW
C = 32
NCHUNK = ROWS_PER_W // C

_mesh = plsc.VectorSubcoreMesh(core_axis_name="c", subcore_axis_name="s")


@functools.partial(
    pl.kernel,
    mesh=_mesh,
    out_type=jax.ShapeDtypeStruct((N, D), jnp.float32),
    scratch_types=[
        pltpu.VMEM((NCHUNK, C), jnp.int32),
        pltpu.VMEM((C, D), jnp.float32),
        pltpu.VMEM((C, D), jnp.float32),
        pltpu.VMEM_SHARED((V, D), jnp.float32),
        pltpu.SemaphoreType.DMA,
        pltpu.SemaphoreType.DMA,
    ],
)
def _sc_gather(idx_hbm, table_hbm, out_hbm, idx_v, rows0, rows1, table_sh,
               sem0, sem1):
    sid = lax.axis_index("s")
    wid = sid * NC + lax.axis_index("c")

    @pl.when(sid == 0)
    def _():
        pltpu.sync_copy(table_hbm, table_sh)

    pltpu.sync_copy(idx_hbm.at[wid], idx_v)
    plsc.subcore_barrier()

    base = wid * ROWS_PER_W
    bufs = (rows0, rows1)
    sems = (sem0, sem1)

    def issue(c, buf, sem):
        for g in range(C // 16):
            vec = idx_v[c, pl.ds(g * 16, 16)]
            for j in range(16):
                r = vec[j]
                pltpu.async_copy(table_sh.at[r], buf.at[g * 16 + j], sem)

    def wait_all(buf, sem):
        pltpu.make_async_copy(table_hbm.at[pl.ds(0, C)], buf, sem).wait()

    issue(0, rows0, sem0)

    def outer(i2, carry):
        c0 = i2 * 2
        for b in range(2):
            c = c0 + b

            @pl.when(c + 1 < NCHUNK)
            def _():
                issue(c + 1, bufs[1 - b], sems[1 - b])

            wait_all(bufs[b], sems[b])
            pltpu.sync_copy(bufs[b], out_hbm.at[pl.ds(base + c * C, C)])
        return carry

    lax.fori_loop(0, NCHUNK // 2, outer, 0)


def kernel(input_ids, attention_mask, embed):
    idx = input_ids.reshape(NW, NCHUNK, C).astype(jnp.int32)
    out = _sc_gather(idx, embed)
    return out.reshape(B, S, D)
